# baseline (speedup 1.0000x reference)
import functools

import jax
import jax.numpy as jnp
from jax import lax
from jax.experimental import pallas as pl
from jax.experimental.pallas import tpu as pltpu
from jax.experimental.pallas import tpu_sc as plsc

N_PAIRS = 1_600_000
N_ATOMS = 50_000
CD = 16

NC = 2
NS = 16
E_SC = N_PAIRS // NC
DUMMY = N_ATOMS

B = 640
NBLK_G = E_SC // B
NBLK_T = 81
ACC_ROWS = 50_048
W = 3128

_mesh = plsc.VectorSubcoreMesh(
    core_axis_name="c", subcore_axis_name="s", num_cores=NC, num_subcores=NS)


@functools.partial(
    pl.kernel,
    out_type=jax.ShapeDtypeStruct((NC, N_ATOMS, CD), jnp.float32),
    mesh=_mesh,
    scratch_types=[
        pltpu.VMEM((B,), jnp.int32),
        pltpu.VMEM((B,), jnp.int32),
        pltpu.VMEM((B,), jnp.int32),
        pltpu.VMEM((B, CD), jnp.float32),
        pltpu.VMEM((B, CD), jnp.float32),
        pltpu.VMEM((B, CD), jnp.float32),
        pltpu.VMEM((B,), jnp.int32),
        pltpu.VMEM((B,), jnp.int32),
        pltpu.VMEM((B,), jnp.int32),
        pltpu.VMEM_SHARED((ACC_ROWS, CD), jnp.float32),
        pltpu.SemaphoreType.DMA,
        pltpu.SemaphoreType.DMA,
        pltpu.SemaphoreType.DMA,
        pltpu.SemaphoreType.DMA,
        pltpu.SemaphoreType.DMA,
        pltpu.SemaphoreType.DMA,
    ],
    compiler_params=pltpu.CompilerParams(use_tc_tiling_on_sc=False),
)
def _scatter_slice(idx_hbm, ixd_hbm, out_hbm,
                   idx_0, idx_1, idx_2, rows_0, rows_1, rows_2,
                   sidx_0, sidx_1, sidx_2, accum_sh,
                   in_sem0, in_sem1, in_sem2, sc_sem0, sc_sem1, sc_sem2):
    idx_bufs = (idx_0, idx_1, idx_2)
    row_bufs = (rows_0, rows_1, rows_2)
    sidx_bufs = (sidx_0, sidx_1, sidx_2)
    in_sems = (in_sem0, in_sem1, in_sem2)
    sc_sems = (sc_sem0, sc_sem1, sc_sem2)
    c = lax.axis_index("c")
    s = lax.axis_index("s")
    e_base = c * E_SC
    zero16 = jnp.zeros((16,), jnp.float32)

    @pl.loop(0, B)
    def _(r):
        rows_0[r, pl.ds(0, 16)] = zero16

    z0 = s * (ACC_ROWS // NS)
    zoff = 0
    while zoff < ACC_ROWS // NS:
        zlen = min(B, ACC_ROWS // NS - zoff)
        pltpu.sync_copy(rows_0.at[pl.ds(0, zlen)],
                        accum_sh.at[pl.ds(z0 + zoff, zlen)])
        zoff += zlen
    plsc.subcore_barrier()

    def in_start(b, q):
        gp = jnp.minimum(s + NS * b, NBLK_G - 1)
        e0 = e_base + gp * B
        pltpu.async_copy(idx_hbm.at[pl.ds(e0, B)], idx_bufs[q], in_sems[q])
        pltpu.async_copy(ixd_hbm.at[pl.ds(e0, B)], row_bufs[q], in_sems[q])

    def in_wait(q):
        pltpu.make_async_copy(idx_hbm.at[pl.ds(0, B)],
                              idx_bufs[q], in_sems[q]).wait()
        pltpu.make_async_copy(ixd_hbm.at[pl.ds(0, B)],
                              row_bufs[q], in_sems[q]).wait()

    def sc_drain(q):
        pltpu.make_async_copy(row_bufs[q],
                              accum_sh.at[sidx_bufs[q]], sc_sems[q]).wait()

    def phase(b, q, drain):
        in_wait(q)
        hi = jnp.where((s + NS * b) < NBLK_G, N_ATOMS, 0)
        idx2 = idx_bufs[q]
        sidx = sidx_bufs[q]

        @pl.loop(0, B // 16)
        def _(i):
            v = idx2[pl.ds(i * 16, 16)]
            si = jnp.where(v < hi, v, DUMMY)
            sidx[pl.ds(i * 16, 16)] = si

        pltpu.async_copy(row_bufs[q], accum_sh.at[sidx], sc_sems[q], add=True)
        if drain:
            sc_drain((q + 2) % 3)
        in_start(b + 2, (q + 2) % 3)

    in_start(0, 0)
    in_start(1, 1)
    phase(0, 0, drain=False)
    phase(1, 1, drain=True)
    phase(2, 2, drain=True)

    @pl.loop(3, NBLK_T, step=3)
    def _(g):
        phase(g, 0, drain=True)
        phase(g + 1, 1, drain=True)
        phase(g + 2, 2, drain=True)

    sc_drain(2)
    in_wait(0)
    in_wait(1)
    plsc.subcore_barrier()

    wstart = jnp.minimum(s * W, N_ATOMS - W)
    pltpu.sync_copy(accum_sh.at[pl.ds(wstart, W)],
                    out_hbm.at[c, pl.ds(wstart, W)])


def kernel(ind_2, px, ix):
    n_atoms = px.shape[0]
    n_pairs, x_dim, c_dim = ix.shape
    idx = ind_2[:, 0]
    parts = [_scatter_slice(idx, ix[:, d, :]) for d in range(x_dim)]
    cols = [p[0] + p[1] for p in parts]
    return jnp.stack(cols, axis=1)


# speedup vs baseline: 1.6300x; 1.6300x over previous
import functools

import jax
import jax.numpy as jnp
from jax import lax
from jax.experimental import pallas as pl
from jax.experimental.pallas import tpu as pltpu
from jax.experimental.pallas import tpu_sc as plsc

N_PAIRS = 1_600_000
N_ATOMS = 50_000
XD = 3
CD = 16
ROW = XD * CD

NC = 2
NS = 16
K = 5
N_PAIRS_C = N_PAIRS // K
E_SC = N_PAIRS_C // NC
DUMMY = N_ATOMS

B = 640
NBLK_G = E_SC // B
NBLK_T = 18
ACC_ROWS = 50_048
W = 3128

_mesh = plsc.VectorSubcoreMesh(
    core_axis_name="c", subcore_axis_name="s", num_cores=NC, num_subcores=NS)


def _make_scatter_slice(d):
    @functools.partial(
        pl.kernel,
        out_type=jax.ShapeDtypeStruct((NC, N_ATOMS, CD), jnp.float32),
        mesh=_mesh,
        scratch_types=[
            pltpu.VMEM((B,), jnp.int32),
            pltpu.VMEM((B,), jnp.int32),
            pltpu.VMEM((B,), jnp.int32),
            pltpu.VMEM((B, CD), jnp.float32),
            pltpu.VMEM((B, CD), jnp.float32),
            pltpu.VMEM((B, CD), jnp.float32),
            pltpu.VMEM((B,), jnp.int32),
            pltpu.VMEM((B,), jnp.int32),
            pltpu.VMEM((B,), jnp.int32),
            pltpu.VMEM_SHARED((ACC_ROWS, CD), jnp.float32),
            pltpu.SemaphoreType.DMA,
            pltpu.SemaphoreType.DMA,
            pltpu.SemaphoreType.DMA,
            pltpu.SemaphoreType.DMA,
            pltpu.SemaphoreType.DMA,
            pltpu.SemaphoreType.DMA,
        ],
        compiler_params=pltpu.CompilerParams(use_tc_tiling_on_sc=False),
    )
    def _scatter_slice(idx_hbm, ixf_hbm, out_hbm,
                       idx_0, idx_1, idx_2, rows_0, rows_1, rows_2,
                       sidx_0, sidx_1, sidx_2, accum_sh,
                       in_sem0, in_sem1, in_sem2,
                       sc_sem0, sc_sem1, sc_sem2):
        idx_bufs = (idx_0, idx_1, idx_2)
        row_bufs = (rows_0, rows_1, rows_2)
        sidx_bufs = (sidx_0, sidx_1, sidx_2)
        in_sems = (in_sem0, in_sem1, in_sem2)
        sc_sems = (sc_sem0, sc_sem1, sc_sem2)
        c = lax.axis_index("c")
        s = lax.axis_index("s")
        e_base = c * E_SC
        zero16 = jnp.zeros((16,), jnp.float32)

        @pl.loop(0, B)
        def _(r):
            rows_0[r, pl.ds(0, 16)] = zero16

        z0 = s * (ACC_ROWS // NS)
        zoff = 0
        while zoff < ACC_ROWS // NS:
            zlen = min(B, ACC_ROWS // NS - zoff)
            pltpu.sync_copy(rows_0.at[pl.ds(0, zlen)],
                            accum_sh.at[pl.ds(z0 + zoff, zlen)])
            zoff += zlen
        plsc.subcore_barrier()

        def in_start(b, q):
            gp = jnp.minimum(s + NS * b, NBLK_G - 1)
            e0 = e_base + gp * B
            pltpu.async_copy(idx_hbm.at[pl.ds(e0, B)],
                             idx_bufs[q], in_sems[q])
            pltpu.async_copy(ixf_hbm.at[pl.ds(e0, B), pl.ds(CD * d, CD)],
                             row_bufs[q], in_sems[q])

        def in_wait(q):
            pltpu.make_async_copy(idx_hbm.at[pl.ds(0, B)],
                                  idx_bufs[q], in_sems[q]).wait()
            pltpu.make_async_copy(ixf_hbm.at[pl.ds(0, B), pl.ds(CD * d, CD)],
                                  row_bufs[q], in_sems[q]).wait()

        def sc_drain(q):
            pltpu.make_async_copy(row_bufs[q],
                                  accum_sh.at[sidx_bufs[q]], sc_sems[q]).wait()

        def phase(b, q, drain):
            in_wait(q)
            hi = jnp.where((s + NS * b) < NBLK_G, N_ATOMS, 0)
            idx2 = idx_bufs[q]
            sidx = sidx_bufs[q]

            @pl.loop(0, B // 16)
            def _(i):
                v = idx2[pl.ds(i * 16, 16)]
                si = jnp.where(v < hi, v, DUMMY)
                sidx[pl.ds(i * 16, 16)] = si

            pltpu.async_copy(row_bufs[q], accum_sh.at[sidx],
                             sc_sems[q], add=True)
            if drain:
                sc_drain((q + 2) % 3)
            in_start(b + 2, (q + 2) % 3)

        in_start(0, 0)
        in_start(1, 1)
        phase(0, 0, drain=False)
        phase(1, 1, drain=True)
        phase(2, 2, drain=True)

        @pl.loop(3, NBLK_T, step=3)
        def _(g):
            phase(g, 0, drain=True)
            phase(g + 1, 1, drain=True)
            phase(g + 2, 2, drain=True)

        sc_drain(2)
        in_wait(0)
        in_wait(1)
        plsc.subcore_barrier()

        wstart = jnp.minimum(s * W, N_ATOMS - W)
        pltpu.sync_copy(accum_sh.at[pl.ds(wstart, W)],
                        out_hbm.at[c, pl.ds(wstart, W)])

    return _scatter_slice


_slice_kernels = [_make_scatter_slice(d) for d in range(XD)]


def kernel(ind_2, px, ix):
    n_atoms = px.shape[0]
    n_pairs, x_dim, c_dim = ix.shape
    idx = ind_2[:, 0]
    cols = [None] * x_dim
    for k in range(K):
        sl = slice(k * N_PAIRS_C, (k + 1) * N_PAIRS_C)
        idx_k = idx[sl]
        ixf_k = ix[sl].reshape(N_PAIRS_C, x_dim * c_dim)
        for d in range(x_dim):
            p = _slice_kernels[d](idx_k, ixf_k)
            part = p[0] + p[1]
            cols[d] = part if cols[d] is None else cols[d] + part
    return jnp.stack(cols, axis=1)


# speedup vs baseline: 1.7187x; 1.0544x over previous
import functools

import jax
import jax.numpy as jnp
from jax import lax
from jax.experimental import pallas as pl
from jax.experimental.pallas import tpu as pltpu
from jax.experimental.pallas import tpu_sc as plsc

N_PAIRS = 1_600_000
N_ATOMS = 50_000
XD = 3
CD = 16
ROW = XD * CD

NC = 2
NS = 16
K = 5
N_PAIRS_C = N_PAIRS // K
E_SC = N_PAIRS_C // NC
DUMMY = N_ATOMS

B = 640
NBLK_G = E_SC // B
NBLK_T = 18
ACC_ROWS = 50_048
W = 3128

_mesh = plsc.VectorSubcoreMesh(
    core_axis_name="c", subcore_axis_name="s", num_cores=NC, num_subcores=NS)


def _make_scatter_chunk():
    @functools.partial(
        pl.kernel,
        out_type=jax.ShapeDtypeStruct((XD, NC, N_ATOMS, CD), jnp.float32),
        mesh=_mesh,
        scratch_types=[
            pltpu.VMEM((B,), jnp.int32),
            pltpu.VMEM((B,), jnp.int32),
            pltpu.VMEM((B,), jnp.int32),
            pltpu.VMEM((B, CD), jnp.float32),
            pltpu.VMEM((B, CD), jnp.float32),
            pltpu.VMEM((B, CD), jnp.float32),
            pltpu.VMEM((B,), jnp.int32),
            pltpu.VMEM((B,), jnp.int32),
            pltpu.VMEM((B,), jnp.int32),
            pltpu.VMEM_SHARED((ACC_ROWS, CD), jnp.float32),
            pltpu.SemaphoreType.DMA,
            pltpu.SemaphoreType.DMA,
            pltpu.SemaphoreType.DMA,
            pltpu.SemaphoreType.DMA,
            pltpu.SemaphoreType.DMA,
            pltpu.SemaphoreType.DMA,
        ],
        compiler_params=pltpu.CompilerParams(use_tc_tiling_on_sc=False),
    )
    def _scatter_chunk(idx_hbm, ixf_hbm, out_hbm,
                       idx_0, idx_1, idx_2, rows_0, rows_1, rows_2,
                       sidx_0, sidx_1, sidx_2, accum_sh,
                       in_sem0, in_sem1, in_sem2,
                       sc_sem0, sc_sem1, sc_sem2):
        idx_bufs = (idx_0, idx_1, idx_2)
        row_bufs = (rows_0, rows_1, rows_2)
        sidx_bufs = (sidx_0, sidx_1, sidx_2)
        in_sems = (in_sem0, in_sem1, in_sem2)
        sc_sems = (sc_sem0, sc_sem1, sc_sem2)
        c = lax.axis_index("c")
        s = lax.axis_index("s")
        e_base = c * E_SC
        zero16 = jnp.zeros((16,), jnp.float32)

        def run_slice(d):
            @pl.loop(0, B)
            def _(r):
                rows_0[r, pl.ds(0, 16)] = zero16

            z0 = s * (ACC_ROWS // NS)
            zoff = 0
            while zoff < ACC_ROWS // NS:
                zlen = min(B, ACC_ROWS // NS - zoff)
                pltpu.sync_copy(rows_0.at[pl.ds(0, zlen)],
                                accum_sh.at[pl.ds(z0 + zoff, zlen)])
                zoff += zlen
            plsc.subcore_barrier()

            run_pipeline(d)

            plsc.subcore_barrier()
            wstart = jnp.minimum(s * W, N_ATOMS - W)
            pltpu.sync_copy(accum_sh.at[pl.ds(wstart, W)],
                            out_hbm.at[d, c, pl.ds(wstart, W)])
            plsc.subcore_barrier()

        def run_pipeline(d):
            def in_start(b, q):
                gp = jnp.minimum(s + NS * b, NBLK_G - 1)
                e0 = e_base + gp * B
                pltpu.async_copy(idx_hbm.at[pl.ds(e0, B)],
                                 idx_bufs[q], in_sems[q])
                pltpu.async_copy(ixf_hbm.at[pl.ds(e0, B), pl.ds(CD * d, CD)],
                                 row_bufs[q], in_sems[q])

            def in_wait(q):
                pltpu.make_async_copy(idx_hbm.at[pl.ds(0, B)],
                                      idx_bufs[q], in_sems[q]).wait()
                pltpu.make_async_copy(
                    ixf_hbm.at[pl.ds(0, B), pl.ds(CD * d, CD)],
                    row_bufs[q], in_sems[q]).wait()

            def sc_drain(q):
                pltpu.make_async_copy(
                    row_bufs[q], accum_sh.at[sidx_bufs[q]], sc_sems[q]).wait()

            def phase(b, q, drain):
                in_wait(q)
                hi = jnp.where((s + NS * b) < NBLK_G, N_ATOMS, 0)
                idx2 = idx_bufs[q]
                sidx = sidx_bufs[q]

                @pl.loop(0, B // 16)
                def _(i):
                    v = idx2[pl.ds(i * 16, 16)]
                    si = jnp.where(v < hi, v, DUMMY)
                    sidx[pl.ds(i * 16, 16)] = si

                pltpu.async_copy(row_bufs[q], accum_sh.at[sidx],
                                 sc_sems[q], add=True)
                if drain:
                    sc_drain((q + 2) % 3)
                in_start(b + 2, (q + 2) % 3)

            in_start(0, 0)
            in_start(1, 1)
            phase(0, 0, drain=False)
            phase(1, 1, drain=True)
            phase(2, 2, drain=True)

            @pl.loop(3, NBLK_T, step=3)
            def _(g):
                phase(g, 0, drain=True)
                phase(g + 1, 1, drain=True)
                phase(g + 2, 2, drain=True)

            sc_drain(2)
            in_wait(0)
            in_wait(1)

        for d in range(XD):
            run_slice(d)

    return _scatter_chunk


_chunk_kernel = _make_scatter_chunk()


def kernel(ind_2, px, ix):
    n_atoms = px.shape[0]
    n_pairs, x_dim, c_dim = ix.shape
    idx = ind_2[:, 0]
    acc = None
    for k in range(K):
        sl = slice(k * N_PAIRS_C, (k + 1) * N_PAIRS_C)
        p = _chunk_kernel(idx[sl], ix[sl].reshape(N_PAIRS_C, x_dim * c_dim))
        part = p[:, 0] + p[:, 1]
        acc = part if acc is None else acc + part
    return acc.transpose(1, 0, 2)


# speedup vs baseline: 1.8426x; 1.0721x over previous
import functools

import jax
import jax.numpy as jnp
from jax import lax
from jax.experimental import pallas as pl
from jax.experimental.pallas import tpu as pltpu
from jax.experimental.pallas import tpu_sc as plsc

N_PAIRS = 1_600_000
N_ATOMS = 50_000
ROW = 48

NC = 2
NS = 16
HALF = N_ATOMS // NC
DUMMY = HALF

K = 5
N_PAIRS_C = N_PAIRS // K
B = 256
NBLK_G = N_PAIRS_C // B
NBLK_T = 81
ACC_ROWS = 25_024
ZROWS = ACC_ROWS // NS
W = 1568

_mesh = plsc.VectorSubcoreMesh(
    core_axis_name="c", subcore_axis_name="s", num_cores=NC, num_subcores=NS)


@functools.partial(
    pl.kernel,
    out_type=jax.ShapeDtypeStruct((N_ATOMS, ROW), jnp.float32),
    mesh=_mesh,
    scratch_types=[
        pltpu.VMEM((B,), jnp.int32),
        pltpu.VMEM((B,), jnp.int32),
        pltpu.VMEM((B,), jnp.int32),
        pltpu.VMEM((B, ROW), jnp.float32),
        pltpu.VMEM((B, ROW), jnp.float32),
        pltpu.VMEM((B, ROW), jnp.float32),
        pltpu.VMEM((B,), jnp.int32),
        pltpu.VMEM((B,), jnp.int32),
        pltpu.VMEM((B,), jnp.int32),
        pltpu.VMEM_SHARED((ACC_ROWS, ROW), jnp.float32),
        pltpu.SemaphoreType.DMA,
        pltpu.SemaphoreType.DMA,
        pltpu.SemaphoreType.DMA,
        pltpu.SemaphoreType.DMA,
        pltpu.SemaphoreType.DMA,
        pltpu.SemaphoreType.DMA,
    ],
    compiler_params=pltpu.CompilerParams(use_tc_tiling_on_sc=False),
)
def _scatter_chunk(idx_hbm, ixf_hbm, acc_hbm, outf_hbm,
                 idx_0, idx_1, idx_2, rows_0, rows_1, rows_2,
                 sidx_0, sidx_1, sidx_2, accum_sh,
                 in_sem0, in_sem1, in_sem2, sc_sem0, sc_sem1, sc_sem2):
    idx_bufs = (idx_0, idx_1, idx_2)
    row_bufs = (rows_0, rows_1, rows_2)
    sidx_bufs = (sidx_0, sidx_1, sidx_2)
    in_sems = (in_sem0, in_sem1, in_sem2)
    sc_sems = (sc_sem0, sc_sem1, sc_sem2)
    c = lax.axis_index("c")
    s = lax.axis_index("s")
    lo = c * HALF

    lstart = jnp.minimum(s * W, HALF - W)
    pltpu.sync_copy(acc_hbm.at[pl.ds(c * HALF + lstart, W)],
                    accum_sh.at[pl.ds(lstart, W)])
    plsc.subcore_barrier()

    def in_start(b, q):
        gp = jnp.minimum(s + NS * b, NBLK_G - 1)
        pltpu.async_copy(idx_hbm.at[pl.ds(gp * B, B)], idx_bufs[q], in_sems[q])
        pltpu.async_copy(ixf_hbm.at[pl.ds(gp * B, B)], row_bufs[q], in_sems[q])

    def in_wait(q):
        pltpu.make_async_copy(idx_hbm.at[pl.ds(0, B)],
                              idx_bufs[q], in_sems[q]).wait()
        pltpu.make_async_copy(ixf_hbm.at[pl.ds(0, B)],
                              row_bufs[q], in_sems[q]).wait()

    def sc_drain(q):
        pltpu.make_async_copy(row_bufs[q],
                              accum_sh.at[sidx_bufs[q]], sc_sems[q]).wait()

    def phase(b, q, drain):
        in_wait(q)
        hi = lo + jnp.where((s + NS * b) < NBLK_G, HALF, 0)
        idx2 = idx_bufs[q]
        sidx = sidx_bufs[q]

        @pl.loop(0, B // 16)
        def _(i):
            v = idx2[pl.ds(i * 16, 16)]
            m = (v >= lo) & (v < hi)
            si = jnp.where(m, v - lo, DUMMY)
            sidx[pl.ds(i * 16, 16)] = si

        pltpu.async_copy(row_bufs[q], accum_sh.at[sidx], sc_sems[q], add=True)
        if drain:
            sc_drain((q + 2) % 3)
        in_start(b + 2, (q + 2) % 3)

    in_start(0, 0)
    in_start(1, 1)
    phase(0, 0, drain=False)
    phase(1, 1, drain=True)
    phase(2, 2, drain=True)

    @pl.loop(3, NBLK_T, step=3)
    def _(g):
        phase(g, 0, drain=True)
        phase(g + 1, 1, drain=True)
        phase(g + 2, 2, drain=True)

    sc_drain(2)
    in_wait(0)
    in_wait(1)
    plsc.subcore_barrier()

    wstart = jnp.minimum(s * W, HALF - W)
    pltpu.sync_copy(accum_sh.at[pl.ds(wstart, W)],
                    outf_hbm.at[pl.ds(c * HALF + wstart, W)])


def kernel(ind_2, px, ix):
    n_atoms = px.shape[0]
    n_pairs, x_dim, c_dim = ix.shape
    idx = ind_2[:, 0]
    acc = jnp.zeros((n_atoms, x_dim * c_dim), jnp.float32)
    for k in range(K):
        sl = slice(k * N_PAIRS_C, (k + 1) * N_PAIRS_C)
        acc = _scatter_chunk(idx[sl],
                             ix[sl].reshape(N_PAIRS_C, x_dim * c_dim), acc)
    return acc.reshape(n_atoms, x_dim, c_dim)
